# baseline (speedup 1.0000x reference)
import functools

import jax
import jax.numpy as jnp
from jax import lax
from jax.experimental import pallas as pl
from jax.experimental.pallas import tpu as pltpu
from jax.experimental.pallas import tpu_sc as plsc

NC = 2
NS = 16
NW = NC * NS
LANE = 16
D = 128
DC = D // LANE
LP = 56
NBUF = 4


def _mesh():
    return plsc.VectorSubcoreMesh(
        core_axis_name="c", subcore_axis_name="s",
        num_cores=NC, num_subcores=NS)


_SC_PARAMS = pltpu.CompilerParams(needs_layout_passes=False)


@functools.lru_cache(maxsize=None)
def _ctx_kernel(B, L):
    NB = B // NW
    FB = 16

    row_buf = pltpu.VMEM((LP, D), jnp.float32)

    @functools.partial(
        pl.kernel,
        out_type=jax.ShapeDtypeStruct((B * D,), jnp.float32),
        mesh=_mesh(),
        compiler_params=_SC_PARAMS,
        scratch_types=[
            pltpu.VMEM((NB * LP,), jnp.int32),
            pltpu.VMEM((NB * LP,), jnp.int32),
            [row_buf] * NBUF,
            [row_buf] * NBUF,
            pltpu.VMEM((FB * D,), jnp.float32),
            [pltpu.SemaphoreType.DMA] * NBUF,
            [pltpu.SemaphoreType.DMA] * NBUF,
        ],
    )
    def ctx_kernel(wemb, gemb, wids, gids, out,
                   wid_v, gid_v, wr, gr, ctx_v, ws, gs):
        wid = lax.axis_index("s") * NC + lax.axis_index("c")
        base = wid * NB
        pltpu.sync_copy(wids.at[pl.ds(base * LP, NB * LP)], wid_v)
        pltpu.sync_copy(gids.at[pl.ds(base * LP, NB * LP)], gid_v)

        def start(b, u):
            pltpu.async_copy(wemb.at[wid_v.at[pl.ds(b * LP, LP)]],
                             wr[u], ws[u])
            pltpu.async_copy(gemb.at[gid_v.at[pl.ds(b * LP, LP)]],
                             gr[u], gs[u])

        def wait(u):
            pltpu.make_async_copy(wemb.at[wid_v.at[pl.ds(0, LP)]],
                                  wr[u], ws[u]).wait()
            pltpu.make_async_copy(gemb.at[gid_v.at[pl.ds(0, LP)]],
                                  gr[u], gs[u]).wait()

        def accum(b, u):
            wrows = wr[u]
            grows = gr[u]

            def row_body(r, accs):
                new = list(accs)
                for k in range(2):
                    l = r * 2 + k
                    for c in range(DC):
                        wv = wrows[l, pl.ds(c * LANE, LANE)]
                        gv = grows[l, pl.ds(c * LANE, LANE)]
                        new[c] = new[c] + (wv + gv)
                return tuple(new)

            zeros = tuple(jnp.zeros((LANE,), jnp.float32) for _ in range(DC))
            accs = lax.fori_loop(0, L // 2, row_body, zeros)
            slot = lax.rem(b, FB)
            for c in range(DC):
                ctx_v[pl.ds(slot * D + c * LANE, LANE)] = accs[c]

        for u in range(NBUF - 1):
            start(u, u)

        def group_body(gi, carry):
            for u in range(NBUF):
                b = gi * NBUF + u

                @pl.when(b + NBUF - 1 < NB)
                def _():
                    start(b + NBUF - 1, (u + NBUF - 1) % NBUF)

                wait(u)
                accum(b, u)

                @pl.when(lax.rem(b, FB) == FB - 1)
                def _():
                    pltpu.sync_copy(
                        ctx_v,
                        out.at[pl.ds((base + b - (FB - 1)) * D, FB * D)])

            return carry

        lax.fori_loop(0, NB // NBUF, group_body, 0)

    return ctx_kernel


@functools.lru_cache(maxsize=None)
def _scores_kernel(B, L):
    NB = B // NW
    NB2 = NB // 2
    FS = 64
    UN = 5
    NRG = (L + LANE - 1) // LANE
    TSTRIDE = LANE + 1

    row_buf = pltpu.VMEM((LP, D), jnp.float32)

    @functools.partial(
        pl.kernel,
        out_type=jax.ShapeDtypeStruct((B * L,), jnp.float32),
        mesh=_mesh(),
        compiler_params=_SC_PARAMS,
        scratch_types=[
            pltpu.VMEM((NB * LP,), jnp.int32),
            pltpu.VMEM((NB2 * D,), jnp.float32),
            [row_buf] * NBUF,
            pltpu.VMEM((FS * L,), jnp.float32),
            [pltpu.SemaphoreType.DMA] * NBUF,
        ],
    )
    def scores_kernel(eemb, eids, projf, out,
                      eid_v, proj_v, er, sc_v, es):
        wid = lax.axis_index("s") * NC + lax.axis_index("c")
        base = wid * NB
        pltpu.sync_copy(eids.at[pl.ds(base * LP, NB * LP)], eid_v)
        pltpu.sync_copy(projf.at[pl.ds(base * D, NB2 * D)], proj_v)

        def start(b, u):
            pltpu.async_copy(eemb.at[eid_v.at[pl.ds(b * LP, LP)]],
                             er[u], es[u])

        def wait(u):
            pltpu.make_async_copy(eemb.at[eid_v.at[pl.ds(0, LP)]],
                                  er[u], es[u]).wait()

        lane = lax.iota(jnp.int32, LANE)
        last_lane = lane == (LANE - 1)

        def dot(b, u):
            rows = er[u]
            poff = lax.rem(b, NB2) * D
            pvs = [proj_v[pl.ds(poff + c * LANE, LANE)] for c in range(DC)]
            slot_base = lax.rem(b, FS) * L

            def row_body(r, carry):
                for k in range(UN):
                    l = r * UN + k
                    prods = [rows[l, pl.ds(c * LANE, LANE)] * pvs[c]
                             for c in range(DC)]
                    while len(prods) > 1:
                        prods = [prods[i] + prods[i + 1]
                                 for i in range(0, len(prods), 2)]
                    cum = plsc.cumsum(prods[0])
                    idx = jnp.full((LANE,), slot_base + l, jnp.int32)
                    plsc.store_scatter(sc_v, [idx], cum, mask=last_lane)
                return carry

            lax.fori_loop(0, L // UN, row_body, 0)

        for u in range(NBUF - 1):
            start(u, u)

        def group_body(gi, carry):
            for u in range(NBUF):
                b = gi * NBUF + u

                @pl.when(b == NB2)
                def _():
                    pltpu.sync_copy(
                        projf.at[pl.ds((base + NB2) * D, NB2 * D)], proj_v)

                @pl.when(b + NBUF - 1 < NB)
                def _():
                    start(b + NBUF - 1, (u + NBUF - 1) % NBUF)

                wait(u)
                dot(b, u)

                @pl.when(lax.rem(b, FS) == FS - 1)
                def _():
                    off = pl.multiple_of((base + b - (FS - 1)) * L, 8)
                    pltpu.sync_copy(sc_v, out.at[pl.ds(off, FS * L)])

            return carry

        lax.fori_loop(0, NB // NBUF, group_body, 0)

    return scores_kernel


def _tc_proj(ctx_sum, W, bias, L):
    B = ctx_sum.shape[0]
    BLK = 2048

    def body(x_ref, w_ref, b_ref, o_ref):
        x = x_ref[...] * (1.0 / L)
        y = lax.dot_general(x, w_ref[...], (((1,), (1,)), ((), ())),
                            preferred_element_type=jnp.float32)
        o_ref[...] = jnp.tanh(y + b_ref[...])

    return pl.pallas_call(
        body,
        grid=(B // BLK,),
        in_specs=[
            pl.BlockSpec((BLK, D), lambda i: (i, 0)),
            pl.BlockSpec((D, D), lambda i: (0, 0)),
            pl.BlockSpec((1, D), lambda i: (0, 0)),
        ],
        out_specs=pl.BlockSpec((BLK, D), lambda i: (i, 0)),
        out_shape=jax.ShapeDtypeStruct((B, D), jnp.float32),
    )(ctx_sum, W, bias.reshape(1, D))


def kernel(word_embs, ent_embs, gram_embs, W, b, word_ids, ent_ids, gram_ids):
    B, L = word_ids.shape
    pad = LP - L

    def _pad_ids(ids):
        return jnp.concatenate([ids, ids[:, :pad]], axis=1).reshape(-1)

    wip = _pad_ids(word_ids)
    gip = _pad_ids(gram_ids)
    eip = _pad_ids(ent_ids)

    ctx_sum = _ctx_kernel(B, L)(word_embs, gram_embs, wip, gip)
    proj = _tc_proj(ctx_sum.reshape(B, D), W, b, L)
    scores = _scores_kernel(B, L)(ent_embs, eip, proj.reshape(-1))
    return scores.reshape(B, L)


# speedup vs baseline: 1.1549x; 1.1549x over previous
import functools

import jax
import jax.numpy as jnp
from jax import lax
from jax.experimental import pallas as pl
from jax.experimental.pallas import tpu as pltpu
from jax.experimental.pallas import tpu_sc as plsc

NC = 2
NS = 16
NW = NC * NS
LANE = 16
D = 128
DC = D // LANE
LP = 56
NBUF = 4


def _mesh():
    return plsc.VectorSubcoreMesh(
        core_axis_name="c", subcore_axis_name="s",
        num_cores=NC, num_subcores=NS)


_SC_PARAMS = pltpu.CompilerParams(needs_layout_passes=False)


@functools.lru_cache(maxsize=None)
def _ctx_kernel(B, L):
    NB = B // NW
    FB = 16

    row_buf = pltpu.VMEM((LP, D), jnp.float32)

    @functools.partial(
        pl.kernel,
        out_type=jax.ShapeDtypeStruct((B * D,), jnp.float32),
        mesh=_mesh(),
        compiler_params=_SC_PARAMS,
        scratch_types=[
            pltpu.VMEM((NB * LP,), jnp.int32),
            pltpu.VMEM((NB * LP,), jnp.int32),
            [row_buf] * NBUF,
            [row_buf] * NBUF,
            pltpu.VMEM((FB * D,), jnp.float32),
            [pltpu.SemaphoreType.DMA] * NBUF,
            [pltpu.SemaphoreType.DMA] * NBUF,
        ],
    )
    def ctx_kernel(wemb, gemb, wids, gids, out,
                   wid_v, gid_v, wr, gr, ctx_v, ws, gs):
        wid = lax.axis_index("s") * NC + lax.axis_index("c")
        base = wid * NB
        pltpu.sync_copy(wids.at[pl.ds(base * LP, NB * LP)], wid_v)
        pltpu.sync_copy(gids.at[pl.ds(base * LP, NB * LP)], gid_v)

        def start(b, u):
            pltpu.async_copy(wemb.at[wid_v.at[pl.ds(b * LP, LP)]],
                             wr[u], ws[u])
            pltpu.async_copy(gemb.at[gid_v.at[pl.ds(b * LP, LP)]],
                             gr[u], gs[u])

        def wait(u):
            pltpu.make_async_copy(wemb.at[wid_v.at[pl.ds(0, LP)]],
                                  wr[u], ws[u]).wait()
            pltpu.make_async_copy(gemb.at[gid_v.at[pl.ds(0, LP)]],
                                  gr[u], gs[u]).wait()

        def accum(b, u):
            wrows = wr[u]
            grows = gr[u]

            def row_body(r, accs):
                new = list(accs)
                for k in range(2):
                    l = r * 2 + k
                    for c in range(DC):
                        wv = wrows[l, pl.ds(c * LANE, LANE)]
                        gv = grows[l, pl.ds(c * LANE, LANE)]
                        new[c] = new[c] + (wv + gv)
                return tuple(new)

            zeros = tuple(jnp.zeros((LANE,), jnp.float32) for _ in range(DC))
            accs = lax.fori_loop(0, L // 2, row_body, zeros)
            slot = lax.rem(b, FB)
            for c in range(DC):
                ctx_v[pl.ds(slot * D + c * LANE, LANE)] = accs[c]

        for u in range(NBUF - 1):
            start(u, u)

        def group_body(gi, carry):
            for u in range(NBUF):
                b = gi * NBUF + u

                @pl.when(b + NBUF - 1 < NB)
                def _():
                    start(b + NBUF - 1, (u + NBUF - 1) % NBUF)

                wait(u)
                accum(b, u)

                @pl.when(lax.rem(b, FB) == FB - 1)
                def _():
                    pltpu.sync_copy(
                        ctx_v,
                        out.at[pl.ds((base + b - (FB - 1)) * D, FB * D)])

            return carry

        lax.fori_loop(0, NB // NBUF, group_body, 0)

    return ctx_kernel


@functools.lru_cache(maxsize=None)
def _scores_kernel(B, L):
    NB = B // NW
    NB2 = NB // 2
    FS = 64
    UN = 10
    NRG = (L + LANE - 1) // LANE
    TSTRIDE = LANE + 1

    row_buf = pltpu.VMEM((LP, D), jnp.float32)

    @functools.partial(
        pl.kernel,
        out_type=jax.ShapeDtypeStruct((B * L,), jnp.float32),
        mesh=_mesh(),
        compiler_params=_SC_PARAMS,
        scratch_types=[
            pltpu.VMEM((NB * LP,), jnp.int32),
            pltpu.VMEM((NB2 * D,), jnp.float32),
            [row_buf] * NBUF,
            pltpu.VMEM((FS * L,), jnp.float32),
            pltpu.VMEM((64 * (LANE + 1),), jnp.float32),
            [pltpu.SemaphoreType.DMA] * NBUF,
        ],
    )
    def scores_kernel(eemb, eids, projf, out,
                      eid_v, proj_v, er, sc_v, tacc, es):
        wid = lax.axis_index("s") * NC + lax.axis_index("c")
        base = wid * NB
        pltpu.sync_copy(eids.at[pl.ds(base * LP, NB * LP)], eid_v)
        pltpu.sync_copy(projf.at[pl.ds(base * D, NB2 * D)], proj_v)

        def start(b, u):
            pltpu.async_copy(eemb.at[eid_v.at[pl.ds(b * LP, LP)]],
                             er[u], es[u])

        def wait(u):
            pltpu.make_async_copy(eemb.at[eid_v.at[pl.ds(0, LP)]],
                                  er[u], es[u]).wait()

        lane = lax.iota(jnp.int32, LANE)
        lane_ts = lane * TSTRIDE

        def dot(b, u):
            rows = er[u]
            poff = lax.rem(b, NB2) * D
            pvs = [proj_v[pl.ds(poff + c * LANE, LANE)] for c in range(DC)]
            slot_base = lax.rem(b, FS) * L

            def row_body(r, carry):
                for k in range(UN):
                    l = r * UN + k
                    prods = [rows[l, pl.ds(c * LANE, LANE)] * pvs[c]
                             for c in range(DC)]
                    while len(prods) > 1:
                        prods = [prods[i] + prods[i + 1]
                                 for i in range(0, len(prods), 2)]
                    plsc.store_scatter(tacc, [lane + l * TSTRIDE], prods[0])
                return carry

            lax.fori_loop(0, L // UN, row_body, 0)

            for rg in range(NRG):
                rowbase = rg * LANE
                gidx = lane_ts + (rowbase * TSTRIDE)
                terms = [plsc.load_gather(tacc, [gidx + j])
                         for j in range(LANE)]
                while len(terms) > 1:
                    terms = [terms[i] + terms[i + 1]
                             for i in range(0, len(terms), 2)]
                acc = terms[0]
                sidx = lane + (slot_base + rowbase)
                nrow = min(LANE, L - rowbase)
                if nrow == LANE:
                    plsc.store_scatter(sc_v, [sidx], acc)
                else:
                    plsc.store_scatter(sc_v, [sidx], acc, mask=lane < nrow)

        for u in range(NBUF - 1):
            start(u, u)

        def group_body(gi, carry):
            for u in range(NBUF):
                b = gi * NBUF + u

                @pl.when(b == NB2)
                def _():
                    pltpu.sync_copy(
                        projf.at[pl.ds((base + NB2) * D, NB2 * D)], proj_v)

                @pl.when(b + NBUF - 1 < NB)
                def _():
                    start(b + NBUF - 1, (u + NBUF - 1) % NBUF)

                wait(u)
                dot(b, u)

                @pl.when(lax.rem(b, FS) == FS - 1)
                def _():
                    off = pl.multiple_of((base + b - (FS - 1)) * L, 8)
                    pltpu.sync_copy(sc_v, out.at[pl.ds(off, FS * L)])

            return carry

        lax.fori_loop(0, NB // NBUF, group_body, 0)

    return scores_kernel


def _tc_proj(ctx_sum, W, bias, L):
    B = ctx_sum.shape[0]
    BLK = 2048

    def body(x_ref, w_ref, b_ref, o_ref):
        x = x_ref[...] * (1.0 / L)
        y = lax.dot_general(x, w_ref[...], (((1,), (1,)), ((), ())),
                            preferred_element_type=jnp.float32)
        o_ref[...] = jnp.tanh(y + b_ref[...])

    return pl.pallas_call(
        body,
        grid=(B // BLK,),
        in_specs=[
            pl.BlockSpec((BLK, D), lambda i: (i, 0)),
            pl.BlockSpec((D, D), lambda i: (0, 0)),
            pl.BlockSpec((1, D), lambda i: (0, 0)),
        ],
        out_specs=pl.BlockSpec((BLK, D), lambda i: (i, 0)),
        out_shape=jax.ShapeDtypeStruct((B, D), jnp.float32),
    )(ctx_sum, W, bias.reshape(1, D))


def kernel(word_embs, ent_embs, gram_embs, W, b, word_ids, ent_ids, gram_ids):
    B, L = word_ids.shape
    pad = LP - L

    def _pad_ids(ids):
        return jnp.concatenate([ids, ids[:, :pad]], axis=1).reshape(-1)

    wip = _pad_ids(word_ids)
    gip = _pad_ids(gram_ids)
    eip = _pad_ids(ent_ids)

    ctx_sum = _ctx_kernel(B, L)(word_embs, gram_embs, wip, gip)
    proj = _tc_proj(ctx_sum.reshape(B, D), W, b, L)
    scores = _scores_kernel(B, L)(ent_embs, eip, proj.reshape(-1))
    return scores.reshape(B, L)
